# baseline (speedup 1.0000x reference)
import jax
import jax.numpy as jnp
from jax import lax
from jax.experimental import pallas as pl

B, Q, N = 4, 2048, 4096
R = B * Q
BR = 512
NB = R // BR


def _loss_body(x_ref, t_ref, a_ref, o_ref):
    i = pl.program_id(0)
    x = x_ref[...]
    t = t_ref[...]
    e = jnp.exp(x)
    s = jnp.sum(e, axis=1, keepdims=True)
    col = lax.broadcasted_iota(jnp.int32, (1, N), 1)
    mask = col == t
    xt = jnp.sum(jnp.where(mask, x, 0.0), axis=1, keepdims=True)
    at = jnp.sum(jnp.where(mask, a_ref[...], 0.0), axis=1, keepdims=True)
    logp = xt - jnp.log(s)
    p = jnp.exp(logp)
    q1 = 1.0 - p
    contrib = -at * q1 * q1 * logp

    @pl.when(i == 0)
    def _init():
        o_ref[...] = jnp.zeros((1, 1), jnp.float32)

    o_ref[...] += jnp.sum(contrib).reshape(1, 1)


def kernel(inputs, targets, alpha):
    x = inputs.reshape(R, N)
    out = pl.pallas_call(
        _loss_body,
        grid=(NB,),
        in_specs=[
            pl.BlockSpec((BR, N), lambda i: (i, 0)),
            pl.BlockSpec((BR, 1), lambda i: (i, 0)),
            pl.BlockSpec((1, N), lambda i: (0, 0)),
        ],
        out_specs=pl.BlockSpec((1, 1), lambda i: (0, 0)),
        out_shape=jax.ShapeDtypeStruct((1, 1), jnp.float32),
    )(x, targets.reshape(R, 1), alpha.reshape(1, N))
    return out[0, 0] / jnp.float32(R)
